# baseline (speedup 1.0000x reference)
import functools

import jax
import jax.numpy as jnp
import numpy as np
from jax import lax
from jax.experimental import pallas as pl
from jax.experimental.pallas import tpu as pltpu
from jax.experimental.pallas import tpu_sc as plsc

TOP_K = 32
N_ROWS = 128
VOCAB = 100000
PV = 102400
NBLK = PV // 128
NW = 32
ROWS_PER_W = N_ROWS // NW
OCH = 10000
NOCH = VOCAB // OCH

_NEG_INF = np.float32(-np.inf)


def _take16(v, idx):
    dnums = lax.GatherDimensionNumbers(
        offset_dims=(), collapsed_slice_dims=(0,), start_index_map=(0,))
    return lax.gather(v, idx[:, None], dnums, slice_sizes=(1,),
                      mode=lax.GatherScatterMode.PROMISE_IN_BOUNDS)


def _splat_f(s):
    return jnp.full((16,), s, jnp.float32)


def _splat_i(s):
    return jnp.full((16,), s, jnp.int32)


def _tourn(v, op):
    lane = lax.iota(jnp.int32, 16)
    for d in (1, 2, 4, 8):
        v = op(v, _take16(v, lane ^ d))
    return v


def _insert_one(st, xv, xi):
    a0v, a0i, a1v, a1i = st
    lane = lax.iota(jnp.int32, 16)
    one = _splat_i(1)
    zero = _splat_i(0)

    def beats(av, ai):
        gt = jnp.where(av > xv, one, zero)
        eq = jnp.where(av == xv, one, zero) * jnp.where(ai < xi, one, zero)
        return gt + eq

    pos = _tourn(beats(a0v, a0i) + beats(a1v, a1i), jnp.add)
    shift = jnp.maximum(lane - 1, 0)
    s0v = _take16(a0v, shift)
    s0i = _take16(a0i, shift)
    last = _splat_i(15)
    bv = _take16(a0v, last)
    bi = _take16(a0i, last)
    t1v = _take16(a1v, shift)
    t1i = _take16(a1i, shift)
    is0 = lane == 0
    s1v = jnp.where(is0, bv, t1v)
    s1i = jnp.where(is0, bi, t1i)
    lt0 = lane < pos
    eq0 = lane == pos
    na0v = jnp.where(lt0, a0v, jnp.where(eq0, xv, s0v))
    na0i = jnp.where(lt0, a0i, jnp.where(eq0, xi, s0i))
    pos1 = pos - 16
    lt1 = lane < pos1
    eq1 = lane == pos1
    na1v = jnp.where(lt1, a1v, jnp.where(eq1, xv, s1v))
    na1i = jnp.where(lt1, a1i, jnp.where(eq1, xi, s1i))
    return (na0v, na0i, na1v, na1i)


def _scan_vreg(st, thv, v, gbase):
    lane = lax.iota(jnp.int32, 16)
    mi = jnp.where(v >= thv, _splat_i(1), _splat_i(0))
    npend = _tourn(mi, jnp.add)[0]

    def body(_, c):
        mi, st4 = c[0], c[1:]
        f = _tourn(jnp.where(mi > 0, lane, _splat_i(16)), jnp.minimum)
        xv = _take16(v, f)
        xi = f + gbase
        st4 = _insert_one(st4, xv, xi)
        return (jnp.where(lane == f, _splat_i(0), mi),) + st4

    out = lax.fori_loop(0, npend, body, (mi,) + st)
    st = out[1:]
    return st, _tourn(st[2], jnp.minimum)


def _sc_body(logits, bm, out, bmbuf, dbuf, obuf,
             insem0, outsem0, outsem1):
    wid = lax.axis_index("s") * 2 + lax.axis_index("c")
    minf16 = _splat_f(_NEG_INF)
    lane = lax.iota(jnp.int32, 16)

    def fill_body(i, _):
        obuf[pl.ds(i * 16, 16)] = minf16
        return 0

    lax.fori_loop(0, (2 * OCH) // 16, fill_body, 0)

    def out_dma(row, c, b):
        return pltpu.make_async_copy(
            obuf.at[pl.ds(b * OCH, OCH)],
            out.at[pl.ds(row * VOCAB + c * OCH, OCH)],
            outsem0 if b == 0 else outsem1,
        )

    def scan_block(st4, thv, loadfn, nvec, gbase_fn):
        vs = [loadfn(j) for j in range(nvec)]
        gm = vs[0]
        for j in range(1, nvec):
            gm = jnp.maximum(gm, vs[j])
        hc = jnp.where(gm >= thv, _splat_i(1), _splat_i(0))
        ntrig = _tourn(hc, jnp.bitwise_or)[0]

        def slow(_, carry3):
            st4, thv = carry3[:4], carry3[4]
            for j in range(nvec):
                st4, thv = _scan_vreg(st4, thv, vs[j], gbase_fn(j))
            return st4 + (thv,)

        return lax.fori_loop(0, ntrig, slow, st4 + (thv,))

    def row_body(ri, _):
        row = wid * ROWS_PER_W + ri
        oct8 = lax.div(row, 8)
        rsub = lax.rem(row, 8)

        pltpu.sync_copy(bm.at[pl.ds(row * NBLK, NBLK)], bmbuf)
        init = (minf16, lane + (1 << 30), minf16, lane + (1 << 30) + 16,
                minf16)

        def ga_body(g, carry):
            st4, thv = carry[:4], carry[4]
            gv = g * 160
            return scan_block(
                st4, thv,
                lambda j: bmbuf[pl.ds(g * 160 + 16 * j, 16)],
                10,
                lambda j: _splat_i(16 * j) + gv)

        finA = lax.fori_loop(0, NBLK // 160, ga_body, init)
        b0i, b1i = finA[1], finA[3]

        bids = [b0i[l] for l in range(16)] + [b1i[l] for l in range(16)]
        for l, bid in enumerate(bids):
            pltpu.make_async_copy(
                logits.at[pl.ds(oct8 * 8, 8), pl.ds(bid * 128, 128)],
                dbuf.at[:, pl.ds(l * 128, 128)],
                insem0,
            ).start()

        def drain(l, _):
            pltpu.make_async_copy(
                logits.at[pl.ds(0, 8), pl.ds(0, 128)],
                dbuf.at[:, pl.ds(0, 128)],
                insem0,
            ).wait()
            return 0

        lax.fori_loop(0, 32, drain, 0)

        def blk_body(l, carry):
            st4, thv = carry[:4], carry[4]
            bidv = _take16(jnp.where(l < 16, b0i, b1i),
                           _splat_i(lax.rem(l, 16)))
            return scan_block(
                st4, thv,
                lambda j: dbuf[rsub, pl.ds(l * 128 + 16 * j, 16)],
                8,
                lambda j: bidv * 128 + _splat_i(16 * j))

        finB = lax.fori_loop(0, 32, blk_body, init)
        a0v, a0i, a1v, a1i = finB[:4]

        ent = [(a0v[l], a0i[l]) for l in range(16)]
        ent += [(a1v[l], a1i[l]) for l in range(16)]

        def patch(c, b, restore):
            for ev, ei in ent:
                loc = ei - c * OCH
                inb = jnp.where((loc >= 0) & (loc < OCH), 1, 0)
                loc = jnp.where(inb > 0, loc, 0)
                slot = b * OCH + (loc // 16) * 16
                lanepos = loc - (loc // 16) * 16
                hitl = jnp.where(lane == _splat_i(lanepos),
                                 _splat_i(inb), _splat_i(0))
                w = obuf[pl.ds(slot, 16)]
                val = minf16 if restore else _splat_f(ev)
                obuf[pl.ds(slot, 16)] = jnp.where(hitl > 0, val, w)

        patch(0, 0, False)
        out_dma(row, 0, 0).start()
        patch(1, 1, False)
        out_dma(row, 1, 1).start()

        def opair(c2, _):
            c0 = 2 * c2
            out_dma(row, c0 - 2, 0).wait()
            patch(c0 - 2, 0, True)
            patch(c0, 0, False)
            out_dma(row, c0, 0).start()
            out_dma(row, c0 - 1, 1).wait()
            patch(c0 - 1, 1, True)
            patch(c0 + 1, 1, False)
            out_dma(row, c0 + 1, 1).start()
            return 0

        lax.fori_loop(1, NOCH // 2, opair, 0)
        out_dma(row, NOCH - 2, 0).wait()
        patch(NOCH - 2, 0, True)
        out_dma(row, NOCH - 1, 1).wait()
        patch(NOCH - 1, 1, True)
        return 0

    lax.fori_loop(0, ROWS_PER_W, row_body, 0)


_sc_topk = functools.partial(
    pl.kernel,
    out_type=jax.ShapeDtypeStruct((N_ROWS * VOCAB,), jnp.float32),
    mesh=plsc.VectorSubcoreMesh(core_axis_name="c", subcore_axis_name="s"),
    scratch_types=[
        pltpu.VMEM((NBLK,), jnp.float32),
        pltpu.VMEM((8, 32 * 128), jnp.float32),
        pltpu.VMEM((2 * OCH,), jnp.float32),
        pltpu.SemaphoreType.DMA,
        pltpu.SemaphoreType.DMA,
        pltpu.SemaphoreType.DMA,
    ],
)(_sc_body)


def kernel(hidden, weight):
    vocab_size, hidden_dim = weight.shape
    n = hidden.shape[0]
    logits = jax.lax.dot_general(
        hidden, weight,
        dimension_numbers=(((1,), (1,)), ((), ())),
        preferred_element_type=jnp.float32,
    )
    logits = jnp.concatenate(
        [logits, jnp.full((n, PV - vocab_size), -jnp.inf, jnp.float32)],
        axis=1)
    bm = jnp.max(logits.reshape(n, NBLK, 128), axis=2).reshape(-1)
    return _sc_topk(logits, bm).reshape(n, vocab_size)
